# baseline (speedup 1.0000x reference)
import functools

import jax
import jax.numpy as jnp
from jax import lax
from jax.experimental import pallas as pl
from jax.experimental.pallas import tpu as pltpu
from jax.experimental.pallas import tpu_sc as plsc

NC, NS, LANES = 2, 16, 16
B, N, L, KM, C = 1024, 200, 30, 4, 8
BT = 128
NBT = B // BT
WPT = 4
NT = N // 8
MAXC = (NT + WPT - 1) // WPT

MOTIF_THRESHOLD = 0.01
MOTIF_EFFECT = 2.0
CONFOUNDER_EFFECT = 0.5
BASE_EFFECT = -1.0

_MESH = plsc.VectorSubcoreMesh(
    core_axis_name="c", subcore_axis_name="s", num_cores=NC, num_subcores=NS
)


@functools.partial(
    pl.kernel,
    out_type=(
        jax.ShapeDtypeStruct((B,), jnp.float32),
        jax.ShapeDtypeStruct((B,), jnp.float32),
        jax.ShapeDtypeStruct((B,), jnp.float32),
    ),
    mesh=_MESH,
    compiler_params=pltpu.CompilerParams(
        needs_layout_passes=False, use_tc_tiling_on_sc=True
    ),
    scratch_types=[
        pltpu.VMEM((L, 8, BT), jnp.int32),
        pltpu.VMEM((L, 8, BT), jnp.int32),
        pltpu.VMEM((8, BT), jnp.float32),
        pltpu.VMEM((8, BT), jnp.float32),
        pltpu.VMEM((C, BT), jnp.float32),
        pltpu.VMEM((LANES,), jnp.float32),
        pltpu.VMEM((2 * BT,), jnp.float32),
        pltpu.VMEM((WPT * 2 * BT,), jnp.float32),
        pltpu.VMEM((32,), jnp.float32),
        pltpu.VMEM((32,), jnp.float32),
        pltpu.VMEM((32,), jnp.float32),
        pltpu.VMEM_SHARED((NS, 2 * BT), jnp.float32),
        pltpu.SemaphoreType.DMA,
        pltpu.SemaphoreType.DMA,
    ],
)
def _sc_kernel(
    rep_hbm, cnt_hbm, conf_hbm, motif_hbm,
    lo_hbm, mc_hbm, cc_hbm,
    cell0, cell1, cnt0, cnt1, conf_v, motif_v, stage_v, peer_v,
    lo_v, mc_v, cc_v, shared_v, sem0, sem1,
):
    cid = lax.axis_index("c")
    sid = lax.axis_index("s")
    phase = lax.rem(sid, WPT)
    bt = cid * (NBT // NC) + sid // WPT
    b0 = pl.multiple_of(bt * BT, BT)

    cells = (cell0, cell1)
    cnts = (cnt0, cnt1)
    sems = (sem0, sem1)

    def start_cell(nt):
        par = (nt // WPT) % 2
        buf, cbuf, sem = cells[par], cnts[par], sems[par]
        d0 = pltpu.async_copy(
            rep_hbm.at[:, pl.ds(nt * 8, 8), pl.ds(b0, BT)], buf, sem
        )
        d1 = pltpu.async_copy(
            cnt_hbm.at[pl.ds(nt * 8, 8), pl.ds(b0, BT)], cbuf, sem
        )
        return (d0, d1)

    descs = {}
    for p in range(WPT):
        @pl.when(phase == p)
        def _prologue(p=p):
            descs[p] = start_cell(p)

    pltpu.sync_copy(motif_hbm, motif_v.at[pl.ds(0, KM)])
    pltpu.sync_copy(conf_hbm.at[:, pl.ds(b0, BT)], conf_v)

    motif_vec = motif_v[...]
    tgt = jnp.int32(0)
    ok = jnp.bool_(True)
    for j in range(KM):
        m = motif_vec[j]
        mi = m.astype(jnp.int32)
        ok = ok & (mi.astype(jnp.float32) == m) & (mi >= 0) & (mi < 32)
        tgt = tgt + lax.shift_left(mi, 5 * j)
    tgt = jnp.where(ok, tgt, jnp.int32(1 << 25))

    zf = jnp.zeros((LANES,), jnp.float32)
    zi = jnp.zeros((LANES,), jnp.int32)
    zb = jnp.zeros((LANES,), jnp.bool_)

    for i in range(2 * BT // LANES):
        stage_v[pl.ds(i * LANES, LANES)] = zf

    def compute_cell(par):
        buf, cbuf = cells[par], cnts[par]

        @pl.loop(0, 2 * (BT // LANES), unroll=1)
        def _j(j):
            n0 = lax.shift_right_logical(j, 1)
            bgo = lax.shift_left(j & 1, 6)
            G = 4
            hs = [zi] * G
            accs = [zb] * G
            for l in range(L):
                vs = [buf[l, n0, pl.ds(bgo + g * LANES, LANES)] for g in range(G)]
                for g in range(G):
                    hs[g] = lax.shift_right_logical(hs[g], 5) | lax.shift_left(vs[g], 15)
                if l >= KM - 1:
                    for g in range(G):
                        accs[g] = accs[g] | (hs[g] == tgt)
            for g in range(G):
                cg = cbuf[n0, pl.ds(bgo + g * LANES, LANES)]
                plsc.addupdate(
                    stage_v.at[pl.ds(bgo + g * LANES, LANES)],
                    jnp.where(accs[g], cg, zf),
                )
                plsc.addupdate(
                    stage_v.at[pl.ds(BT + bgo + g * LANES, LANES)], cg
                )

    for k in range(MAXC):
        for p in range(WPT):
            nt = p + WPT * k
            if nt < NT:
                @pl.when(phase == p)
                def _dma(nt=nt):
                    if nt + WPT < NT:
                        descs[nt + WPT] = start_cell(nt + WPT)
                    for d in descs[nt]:
                        d.wait()
        if WPT * k + WPT - 1 < NT:
            compute_cell(k % 2)
        else:
            pl.when(phase == 0)(lambda k=k: compute_cell(k % 2))

    pltpu.sync_copy(stage_v, shared_v.at[sid])
    plsc.subcore_barrier()
    row0 = (sid // WPT) * WPT
    for r in range(WPT):
        pltpu.sync_copy(
            shared_v.at[row0 + r], peer_v.at[pl.ds(r * 2 * BT, 2 * BT)]
        )

    q0 = phase * (BT // WPT)
    for i in range(2):
        off = q0 + i * LANES
        num16 = zf
        den16 = zf
        for r in range(WPT):
            num16 = num16 + peer_v[pl.ds(r * 2 * BT + off, LANES)]
            den16 = den16 + peer_v[pl.ds(r * 2 * BT + BT + off, LANES)]
        mc16 = (num16 > MOTIF_THRESHOLD * den16).astype(jnp.float32)
        mc16 = MOTIF_EFFECT * mc16
        csum = zf
        for c in range(C):
            csum = csum + conf_v[c, pl.ds(off, LANES)]
        cc16 = CONFOUNDER_EFFECT * csum
        lo_v[pl.ds(i * LANES, LANES)] = mc16 + cc16 + BASE_EFFECT
        mc_v[pl.ds(i * LANES, LANES)] = mc16
        cc_v[pl.ds(i * LANES, LANES)] = cc16

    my_b0 = b0 + q0
    pltpu.sync_copy(lo_v, lo_hbm.at[pl.ds(my_b0, 32)])
    pltpu.sync_copy(mc_v, mc_hbm.at[pl.ds(my_b0, 32)])
    pltpu.sync_copy(cc_v, cc_hbm.at[pl.ds(my_b0, 32)])


@jax.jit
def kernel(repertoires, seq_counts, confounds, motif):
    rep_t = jnp.transpose(repertoires, (2, 1, 0))
    cnt_t = jnp.transpose(seq_counts, (1, 0))
    conf_t = jnp.transpose(confounds, (1, 0))
    return _sc_kernel(rep_t, cnt_t, conf_t, motif)


# speedup vs baseline: 1.0264x; 1.0264x over previous
import functools

import jax
import jax.numpy as jnp
from jax import lax
from jax.experimental import pallas as pl
from jax.experimental.pallas import tpu as pltpu
from jax.experimental.pallas import tpu_sc as plsc

NC, NS, LANES = 2, 16, 16
B, N, L, KM, C = 1024, 200, 30, 4, 8
BT = 128
NBT = B // BT
WPT = 4
NT = N // 8
MAXC = (NT + WPT - 1) // WPT

MOTIF_THRESHOLD = 0.01
MOTIF_EFFECT = 2.0
CONFOUNDER_EFFECT = 0.5
BASE_EFFECT = -1.0

_MESH = plsc.VectorSubcoreMesh(
    core_axis_name="c", subcore_axis_name="s", num_cores=NC, num_subcores=NS
)


@functools.partial(
    pl.kernel,
    out_type=(
        jax.ShapeDtypeStruct((B,), jnp.float32),
        jax.ShapeDtypeStruct((B,), jnp.float32),
        jax.ShapeDtypeStruct((B,), jnp.float32),
    ),
    mesh=_MESH,
    compiler_params=pltpu.CompilerParams(
        needs_layout_passes=False, use_tc_tiling_on_sc=True
    ),
    scratch_types=[
        pltpu.VMEM((L, 8, BT), jnp.int32),
        pltpu.VMEM((L, 8, BT), jnp.int32),
        pltpu.VMEM((8, BT), jnp.float32),
        pltpu.VMEM((8, BT), jnp.float32),
        pltpu.VMEM((C, BT), jnp.float32),
        pltpu.VMEM((LANES,), jnp.float32),
        pltpu.VMEM((2 * BT,), jnp.float32),
        pltpu.VMEM((WPT * 2 * BT,), jnp.float32),
        pltpu.VMEM((32,), jnp.float32),
        pltpu.VMEM((32,), jnp.float32),
        pltpu.VMEM((32,), jnp.float32),
        pltpu.VMEM_SHARED((NS, 2 * BT), jnp.float32),
        pltpu.SemaphoreType.DMA,
        pltpu.SemaphoreType.DMA,
    ],
)
def _sc_kernel(
    rep_hbm, cnt_hbm, conf_hbm, motif_hbm,
    lo_hbm, mc_hbm, cc_hbm,
    cell0, cell1, cnt0, cnt1, conf_v, motif_v, stage_v, peer_v,
    lo_v, mc_v, cc_v, shared_v, sem0, sem1,
):
    cid = lax.axis_index("c")
    sid = lax.axis_index("s")
    phase = lax.rem(sid, WPT)
    bt = cid * (NBT // NC) + sid // WPT
    b0 = pl.multiple_of(bt * BT, BT)

    cells = (cell0, cell1)
    cnts = (cnt0, cnt1)
    sems = (sem0, sem1)

    def start_cell(nt):
        par = (nt // WPT) % 2
        buf, cbuf, sem = cells[par], cnts[par], sems[par]
        d0 = pltpu.async_copy(
            rep_hbm.at[:, pl.ds(nt * 8, 8), pl.ds(b0, BT)], buf, sem
        )
        d1 = pltpu.async_copy(
            cnt_hbm.at[pl.ds(nt * 8, 8), pl.ds(b0, BT)], cbuf, sem
        )
        return (d0, d1)

    descs = {}
    for p in range(WPT):
        @pl.when(phase == p)
        def _prologue(p=p):
            descs[p] = start_cell(p)

    pltpu.sync_copy(motif_hbm, motif_v.at[pl.ds(0, KM)])
    pltpu.sync_copy(conf_hbm.at[:, pl.ds(b0, BT)], conf_v)

    motif_vec = motif_v[...]
    tgt = jnp.int32(0)
    ok = jnp.bool_(True)
    for j in range(KM):
        m = motif_vec[j]
        mi = m.astype(jnp.int32)
        ok = ok & (mi.astype(jnp.float32) == m) & (mi >= 0) & (mi < 32)
        tgt = tgt + lax.shift_left(mi, 5 * j)
    tgt = jnp.where(ok, tgt, jnp.int32(1 << 25))

    zf = jnp.zeros((LANES,), jnp.float32)
    zi = jnp.zeros((LANES,), jnp.int32)
    zb = jnp.zeros((LANES,), jnp.bool_)

    for i in range(2 * BT // LANES):
        stage_v[pl.ds(i * LANES, LANES)] = zf

    def compute_cell(par, shared_tail=False):
        buf, cbuf = cells[par], cnts[par]

        hi = 2 * (BT // LANES) if shared_tail else 4 * (BT // LANES)

        @pl.loop(0, hi, unroll=1)
        def _j(j):
            n0 = lax.shift_right_logical(j, 2)
            if shared_tail:
                n0 = n0 + 2 * phase
            bgo = lax.shift_left(j & 3, 5)
            bg1 = bgo + LANES
            ha = zi
            hb = zi
            acca = zb
            accbb = zb
            for l in range(L):
                va = buf[l, n0, pl.ds(bgo, LANES)]
                vb = buf[l, n0, pl.ds(bg1, LANES)]
                ha = lax.shift_right_logical(ha, 5) | lax.shift_left(va, 15)
                hb = lax.shift_right_logical(hb, 5) | lax.shift_left(vb, 15)
                if l >= KM - 1:
                    acca = acca | (ha == tgt)
                    accbb = accbb | (hb == tgt)
            ca = cbuf[n0, pl.ds(bgo, LANES)]
            cb = cbuf[n0, pl.ds(bg1, LANES)]
            plsc.addupdate(
                stage_v.at[pl.ds(bgo, LANES)], jnp.where(acca, ca, zf)
            )
            plsc.addupdate(
                stage_v.at[pl.ds(bg1, LANES)], jnp.where(accbb, cb, zf)
            )
            plsc.addupdate(stage_v.at[pl.ds(BT + bgo, LANES)], ca)
            plsc.addupdate(stage_v.at[pl.ds(BT + bg1, LANES)], cb)

    NK = NT // WPT
    for k in range(NK):
        for p in range(WPT):
            nt = p + WPT * k

            @pl.when(phase == p)
            def _dma(nt=nt, k=k):
                if nt + WPT < WPT * NK:
                    descs[nt + WPT] = start_cell(nt + WPT)
                elif k == NK - 1:
                    descs[NT - 1] = start_cell(NT - 1)
                for d in descs[nt]:
                    d.wait()
        compute_cell(k % 2)
    for d in descs[NT - 1]:
        d.wait()
    compute_cell((NT - 1) // WPT % 2, shared_tail=True)

    pltpu.sync_copy(stage_v, shared_v.at[sid])
    plsc.subcore_barrier()
    row0 = (sid // WPT) * WPT
    for r in range(WPT):
        pltpu.sync_copy(
            shared_v.at[row0 + r], peer_v.at[pl.ds(r * 2 * BT, 2 * BT)]
        )

    q0 = phase * (BT // WPT)
    for i in range(2):
        off = q0 + i * LANES
        num16 = zf
        den16 = zf
        for r in range(WPT):
            num16 = num16 + peer_v[pl.ds(r * 2 * BT + off, LANES)]
            den16 = den16 + peer_v[pl.ds(r * 2 * BT + BT + off, LANES)]
        mc16 = (num16 > MOTIF_THRESHOLD * den16).astype(jnp.float32)
        mc16 = MOTIF_EFFECT * mc16
        csum = zf
        for c in range(C):
            csum = csum + conf_v[c, pl.ds(off, LANES)]
        cc16 = CONFOUNDER_EFFECT * csum
        lo_v[pl.ds(i * LANES, LANES)] = mc16 + cc16 + BASE_EFFECT
        mc_v[pl.ds(i * LANES, LANES)] = mc16
        cc_v[pl.ds(i * LANES, LANES)] = cc16

    my_b0 = b0 + q0
    pltpu.sync_copy(lo_v, lo_hbm.at[pl.ds(my_b0, 32)])
    pltpu.sync_copy(mc_v, mc_hbm.at[pl.ds(my_b0, 32)])
    pltpu.sync_copy(cc_v, cc_hbm.at[pl.ds(my_b0, 32)])


@jax.jit
def kernel(repertoires, seq_counts, confounds, motif):
    rep_t = jnp.transpose(repertoires, (2, 1, 0))
    cnt_t = jnp.transpose(seq_counts, (1, 0))
    conf_t = jnp.transpose(confounds, (1, 0))
    return _sc_kernel(rep_t, cnt_t, conf_t, motif)


# speedup vs baseline: 1.0378x; 1.0111x over previous
import functools

import jax
import jax.numpy as jnp
from jax import lax
from jax.experimental import pallas as pl
from jax.experimental.pallas import tpu as pltpu
from jax.experimental.pallas import tpu_sc as plsc

NC, NS, LANES = 2, 16, 16
B, N, L, KM, C = 1024, 200, 30, 4, 8
BT = 128
NBT = B // BT
WPT = 4
NT = N // 8
MAXC = (NT + WPT - 1) // WPT

MOTIF_THRESHOLD = 0.01
MOTIF_EFFECT = 2.0
CONFOUNDER_EFFECT = 0.5
BASE_EFFECT = -1.0

_MESH = plsc.VectorSubcoreMesh(
    core_axis_name="c", subcore_axis_name="s", num_cores=NC, num_subcores=NS
)


@functools.partial(
    pl.kernel,
    out_type=(
        jax.ShapeDtypeStruct((B,), jnp.float32),
        jax.ShapeDtypeStruct((B,), jnp.float32),
        jax.ShapeDtypeStruct((B,), jnp.float32),
    ),
    mesh=_MESH,
    compiler_params=pltpu.CompilerParams(
        needs_layout_passes=False, use_tc_tiling_on_sc=True
    ),
    scratch_types=[
        pltpu.VMEM((L, 8, BT), jnp.int32),
        pltpu.VMEM((L, 8, BT), jnp.int32),
        pltpu.VMEM((8, BT), jnp.float32),
        pltpu.VMEM((8, BT), jnp.float32),
        pltpu.VMEM((C, BT), jnp.float32),
        pltpu.VMEM((LANES,), jnp.float32),
        pltpu.VMEM((2 * BT,), jnp.float32),
        pltpu.VMEM((WPT * 2 * BT,), jnp.float32),
        pltpu.VMEM((32,), jnp.float32),
        pltpu.VMEM((32,), jnp.float32),
        pltpu.VMEM((32,), jnp.float32),
        pltpu.VMEM_SHARED((NS, 2 * BT), jnp.float32),
        pltpu.SemaphoreType.DMA,
        pltpu.SemaphoreType.DMA,
    ],
)
def _sc_kernel(
    rep_hbm, cnt_hbm, conf_hbm, motif_hbm,
    lo_hbm, mc_hbm, cc_hbm,
    cell0, cell1, cnt0, cnt1, conf_v, motif_v, stage_v, peer_v,
    lo_v, mc_v, cc_v, shared_v, sem0, sem1,
):
    cid = lax.axis_index("c")
    sid = lax.axis_index("s")
    phase = lax.rem(sid, WPT)
    bt = cid * (NBT // NC) + sid // WPT
    b0 = pl.multiple_of(bt * BT, BT)

    cells = (cell0, cell1)
    cnts = (cnt0, cnt1)
    sems = (sem0, sem1)

    def start_cell(nt):
        par = (nt // WPT) % 2
        buf, cbuf, sem = cells[par], cnts[par], sems[par]
        d0 = pltpu.async_copy(
            rep_hbm.at[:, pl.ds(nt * 8, 8), pl.ds(b0, BT)], buf, sem
        )
        d1 = pltpu.async_copy(
            cnt_hbm.at[pl.ds(nt * 8, 8), pl.ds(b0, BT)], cbuf, sem
        )
        return (d0, d1)

    descs = {}
    for p in range(WPT):
        @pl.when(phase == p)
        def _prologue(p=p):
            descs[p] = start_cell(p)

    pltpu.sync_copy(motif_hbm, motif_v.at[pl.ds(0, KM)])
    pltpu.sync_copy(conf_hbm.at[:, pl.ds(b0, BT)], conf_v)

    motif_vec = motif_v[...]
    tgt = jnp.int32(0)
    ok = jnp.bool_(True)
    for j in range(KM):
        m = motif_vec[j]
        mi = m.astype(jnp.int32)
        ok = ok & (mi.astype(jnp.float32) == m) & (mi >= 0) & (mi < 32)
        tgt = tgt + lax.shift_left(mi, 5 * j)
    tgt = jnp.where(ok, tgt, jnp.int32(1 << 25))

    zf = jnp.zeros((LANES,), jnp.float32)
    zi = jnp.zeros((LANES,), jnp.int32)
    zb = jnp.zeros((LANES,), jnp.bool_)

    for i in range(2 * BT // LANES):
        stage_v[pl.ds(i * LANES, LANES)] = zf

    def compute_cell(par, shared_tail=False):
        buf, cbuf = cells[par], cnts[par]

        hi = BT // LANES if shared_tail else 4 * (BT // LANES)

        @pl.loop(0, hi, unroll=1)
        def _j(j):
            n0 = lax.shift_right_logical(j, 2)
            if shared_tail:
                n0 = n0 + 2 * phase
            bgo = lax.shift_left(j & 3, 5)
            bg1 = bgo + LANES
            ha = zi
            hb = zi
            acca = zb
            accbb = zb
            for l in range(L):
                va = buf[l, n0, pl.ds(bgo, LANES)]
                vb = buf[l, n0, pl.ds(bg1, LANES)]
                ha = lax.shift_right_logical(ha, 5) | lax.shift_left(va, 15)
                hb = lax.shift_right_logical(hb, 5) | lax.shift_left(vb, 15)
                if l >= KM - 1:
                    acca = acca | (ha == tgt)
                    accbb = accbb | (hb == tgt)
            ca = cbuf[n0, pl.ds(bgo, LANES)]
            cb = cbuf[n0, pl.ds(bg1, LANES)]
            plsc.addupdate(
                stage_v.at[pl.ds(bgo, LANES)], jnp.where(acca, ca, zf)
            )
            plsc.addupdate(
                stage_v.at[pl.ds(bg1, LANES)], jnp.where(accbb, cb, zf)
            )
            plsc.addupdate(stage_v.at[pl.ds(BT + bgo, LANES)], ca)
            plsc.addupdate(stage_v.at[pl.ds(BT + bg1, LANES)], cb)

    NK = NT // WPT
    for k in range(NK):
        for p in range(WPT):
            nt = p + WPT * k

            @pl.when(phase == p)
            def _dma(nt=nt, k=k):
                if nt + WPT < WPT * NK:
                    descs[nt + WPT] = start_cell(nt + WPT)
                elif k == NK - 1:
                    descs[NT - 1] = start_cell(NT - 1)
                for d in descs[nt]:
                    d.wait()
        compute_cell(k % 2)
    for d in descs[NT - 1]:
        d.wait()
    compute_cell((NT - 1) // WPT % 2, shared_tail=True)

    pltpu.sync_copy(stage_v, shared_v.at[sid])
    plsc.subcore_barrier()
    row0 = (sid // WPT) * WPT
    for r in range(WPT):
        pltpu.sync_copy(
            shared_v.at[row0 + r], peer_v.at[pl.ds(r * 2 * BT, 2 * BT)]
        )

    q0 = phase * (BT // WPT)
    for i in range(2):
        off = q0 + i * LANES
        num16 = zf
        den16 = zf
        for r in range(WPT):
            num16 = num16 + peer_v[pl.ds(r * 2 * BT + off, LANES)]
            den16 = den16 + peer_v[pl.ds(r * 2 * BT + BT + off, LANES)]
        mc16 = (num16 > MOTIF_THRESHOLD * den16).astype(jnp.float32)
        mc16 = MOTIF_EFFECT * mc16
        csum = zf
        for c in range(C):
            csum = csum + conf_v[c, pl.ds(off, LANES)]
        cc16 = CONFOUNDER_EFFECT * csum
        lo_v[pl.ds(i * LANES, LANES)] = mc16 + cc16 + BASE_EFFECT
        mc_v[pl.ds(i * LANES, LANES)] = mc16
        cc_v[pl.ds(i * LANES, LANES)] = cc16

    my_b0 = b0 + q0
    pltpu.sync_copy(lo_v, lo_hbm.at[pl.ds(my_b0, 32)])
    pltpu.sync_copy(mc_v, mc_hbm.at[pl.ds(my_b0, 32)])
    pltpu.sync_copy(cc_v, cc_hbm.at[pl.ds(my_b0, 32)])


@jax.jit
def kernel(repertoires, seq_counts, confounds, motif):
    rep_t = jnp.transpose(repertoires, (2, 1, 0))
    cnt_t = jnp.transpose(seq_counts, (1, 0))
    conf_t = jnp.transpose(confounds, (1, 0))
    return _sc_kernel(rep_t, cnt_t, conf_t, motif)
